# Initial kernel scaffold; baseline (speedup 1.0000x reference)
import jax
import jax.numpy as jnp
from jax.experimental import pallas as pl


def kernel(token_ids, position_ids, W_word, W_pos):
    raise NotImplementedError("write your pallas kernel here")


# speedup vs baseline: 3.0033x; 3.0033x over previous
import functools

import jax
import jax.numpy as jnp
from jax import lax
from jax.experimental import pallas as pl
from jax.experimental.pallas import tpu as pltpu
from jax.experimental.pallas import tpu_sc as plsc

L = 16
C = 16
D = 64
VOCAB = 1000
NPOS = 16
NW = 32
ROWS = 1024 * 50
ROWS_PER_TILE = ROWS // NW
CHUNK = 320
NCHUNK = ROWS_PER_TILE // CHUNK
BLOCKS = CHUNK // L


def _sc_body(tok_hbm, pos_hbm, wword_hbm, wpos_hbm, out_hbm,
             wword_v, wpos_v, tok_v, pos_v, out_v):
    wid = lax.axis_index("s") * 2 + lax.axis_index("c")
    base = wid * ROWS_PER_TILE

    pltpu.sync_copy(wword_hbm, wword_v)
    pltpu.sync_copy(wpos_hbm, wpos_v)

    riota = lax.broadcasted_iota(jnp.int32, (L,), 0)

    for chunk in range(NCHUNK):
        r0 = base + chunk * CHUNK
        pltpu.sync_copy(tok_hbm.at[pl.ds(r0 * C, CHUNK * C)], tok_v)
        pltpu.sync_copy(pos_hbm.at[pl.ds(r0 * C, CHUNK * C)], pos_v)

        def block_body(b, carry):
            rb = b * L + riota
            rb_c = rb * C
            rb_d = rb * D
            tok_c = [plsc.load_gather(tok_v, [rb_c + c]) * D for c in range(C)]
            pos_c = [plsc.load_gather(pos_v, [rb_c + c]) * D for c in range(C)]

            def jbody(j, carry2):
                acc = plsc.load_gather(wword_v, [tok_c[0] + j])
                for c in range(1, C):
                    acc = acc + plsc.load_gather(wword_v, [tok_c[c] + j])
                for c in range(C):
                    acc = acc + plsc.load_gather(wpos_v, [pos_c[c] + j])
                plsc.store_scatter(out_v, [rb_d + j], acc)
                return carry2

            return lax.fori_loop(0, D, jbody, carry)

        lax.fori_loop(0, BLOCKS, block_body, 0)
        pltpu.sync_copy(out_v, out_hbm.at[pl.ds(r0 * D, CHUNK * D)])


@functools.partial(jax.jit, static_argnames=())
def kernel(token_ids, position_ids, W_word, W_pos):
    B, W, _ = token_ids.shape
    tok = token_ids.reshape(ROWS * C).astype(jnp.int32)
    pos = position_ids.reshape(ROWS * C).astype(jnp.int32)

    mesh = plsc.VectorSubcoreMesh(core_axis_name="c", subcore_axis_name="s")
    out = pl.kernel(
        _sc_body,
        out_type=jax.ShapeDtypeStruct((ROWS * D,), jnp.float32),
        mesh=mesh,
        compiler_params=pltpu.CompilerParams(needs_layout_passes=False),
        scratch_types=[
            pltpu.VMEM((VOCAB * D,), jnp.float32),
            pltpu.VMEM((NPOS * D,), jnp.float32),
            pltpu.VMEM((CHUNK * C,), jnp.int32),
            pltpu.VMEM((CHUNK * C,), jnp.int32),
            pltpu.VMEM((CHUNK * D,), jnp.float32),
        ],
    )(tok, pos, W_word.reshape(VOCAB * D), W_pos.reshape(NPOS * D))
    return out.reshape(B, W, D)
